# initial kernel scaffold (unmeasured)
import functools

import jax
import jax.numpy as jnp
from jax import lax
from jax.experimental import pallas as pl
from jax.experimental.pallas import tpu as pltpu

N_DEV = 8


def kernel(x, w_mat):
    m, k_per = x.shape
    _, n = w_mat.shape
    m_per = m // N_DEV

    def body(x_ref, w_ref, out_ref, buf, send_sems, recv_sems, credit_sem):
        d = lax.axis_index("i")
        left = lax.rem(d + N_DEV - 1, N_DEV)
        right = lax.rem(d + 1, N_DEV)

        barrier = pltpu.get_barrier_semaphore()
        for nbr in (left, right):
            pl.semaphore_signal(
                barrier, inc=1, device_id=(nbr,),
                device_id_type=pl.DeviceIdType.MESH,
            )
        pl.semaphore_wait(barrier, 2)

        def chunk_gemm(c):
            xc = x_ref[pl.ds(c * m_per, m_per), :]
            return jnp.dot(xc, w_ref[...], preferred_element_type=jnp.float32)

        for h in range(N_DEV - 1):
            s = h % 2
            r = (h + 1) % 2
            c = lax.rem(d + (N_DEV - 1) - h, N_DEV)
            if h == 0:
                buf[s] = chunk_gemm(c)
            else:
                buf[s] = buf[s] + chunk_gemm(c)
            rdma = pltpu.make_async_remote_copy(
                src_ref=buf.at[s],
                dst_ref=buf.at[r],
                send_sem=send_sems.at[s],
                recv_sem=recv_sems.at[r],
                device_id=(right,),
                device_id_type=pl.DeviceIdType.MESH,
            )
            if h >= 1:
                pl.semaphore_wait(credit_sem, 1)
            rdma.start()
            rdma.wait()
            if h < N_DEV - 2:
                pl.semaphore_signal(
                    credit_sem, inc=1, device_id=(left,),
                    device_id_type=pl.DeviceIdType.MESH,
                )

        out_ref[...] = buf[1] + chunk_gemm(d)

        @functools.partial(
            pl.run_scoped, second_barrier=pltpu.SemaphoreType.REGULAR
        )
        def _(second_barrier):
            for nbr in (left, right):
                pl.semaphore_signal(
                    second_barrier, inc=1, device_id=(nbr,),
                    device_id_type=pl.DeviceIdType.MESH,
                )
            pl.semaphore_wait(second_barrier, 2)

    return pl.pallas_call(
        body,
        out_shape=jax.ShapeDtypeStruct((m_per, n), jnp.float32),
        in_specs=[
            pl.BlockSpec(memory_space=pltpu.VMEM),
            pl.BlockSpec(memory_space=pltpu.VMEM),
        ],
        out_specs=pl.BlockSpec(memory_space=pltpu.VMEM),
        scratch_shapes=[
            pltpu.VMEM((2, m_per, n), jnp.float32),
            pltpu.SemaphoreType.DMA((2,)),
            pltpu.SemaphoreType.DMA((2,)),
            pltpu.SemaphoreType.REGULAR,
        ],
        compiler_params=pltpu.CompilerParams(collective_id=0),
    )(x, w_mat)


# baseline (device time: 1343196 ns/iter reference)
import functools

import jax
import jax.numpy as jnp
from jax import lax
from jax.experimental import pallas as pl
from jax.experimental.pallas import tpu as pltpu

N_DEV = 8


def kernel(x, w_mat):
    m, k_per = x.shape
    _, n = w_mat.shape
    m_per = m // N_DEV

    def body(x_ref, w_ref, out_ref, buf, send_sems, recv_sems, credit_sem,
             out_copy_sem):
        d = lax.axis_index("i")
        left = lax.rem(d + N_DEV - 1, N_DEV)
        right = lax.rem(d + 1, N_DEV)

        barrier = pltpu.get_barrier_semaphore()
        for nbr in (left, right):
            pl.semaphore_signal(
                barrier, inc=1, device_id=(nbr,),
                device_id_type=pl.DeviceIdType.MESH,
            )
        pl.semaphore_wait(barrier, 2)

        def chunk_gemm(c):
            xc = x_ref[pl.ds(c * m_per, m_per), :]
            return jnp.dot(xc, w_ref[...], preferred_element_type=jnp.float32)

        for h in range(N_DEV - 1):
            s = h % 2
            r = (h + 1) % 2
            c = lax.rem(d + (N_DEV - 1) - h, N_DEV)
            if h == 0:
                buf[s] = chunk_gemm(c)
            else:
                buf[s] = buf[s] + chunk_gemm(c)
            rdma = pltpu.make_async_remote_copy(
                src_ref=buf.at[s],
                dst_ref=buf.at[r],
                send_sem=send_sems.at[s],
                recv_sem=recv_sems.at[r],
                device_id=(right,),
                device_id_type=pl.DeviceIdType.MESH,
            )
            if h >= 1:
                pl.semaphore_wait(credit_sem, 1)
            rdma.start()
            rdma.wait()
            if h < N_DEV - 2:
                pl.semaphore_signal(
                    credit_sem, inc=1, device_id=(left,),
                    device_id_type=pl.DeviceIdType.MESH,
                )

        buf[0] = buf[1] + chunk_gemm(d)
        out_copy = pltpu.make_async_copy(buf.at[0], out_ref, out_copy_sem)
        out_copy.start()
        out_copy.wait()

        @functools.partial(
            pl.run_scoped, second_barrier=pltpu.SemaphoreType.REGULAR
        )
        def _(second_barrier):
            for nbr in (left, right):
                pl.semaphore_signal(
                    second_barrier, inc=1, device_id=(nbr,),
                    device_id_type=pl.DeviceIdType.MESH,
                )
            pl.semaphore_wait(second_barrier, 2)

    return pl.pallas_call(
        body,
        out_shape=jax.ShapeDtypeStruct((m_per, n), jnp.float32),
        in_specs=[
            pl.BlockSpec(memory_space=pltpu.VMEM),
            pl.BlockSpec(memory_space=pltpu.VMEM),
        ],
        out_specs=pl.BlockSpec(memory_space=pl.ANY),
        scratch_shapes=[
            pltpu.VMEM((2, m_per, n), jnp.float32),
            pltpu.SemaphoreType.DMA((2,)),
            pltpu.SemaphoreType.DMA((2,)),
            pltpu.SemaphoreType.REGULAR,
            pltpu.SemaphoreType.DMA,
        ],
        compiler_params=pltpu.CompilerParams(
            collective_id=0,
            vmem_limit_bytes=63 * 1024 * 1024,
        ),
    )(x, w_mat)


# device time: 716534 ns/iter; 1.8746x vs baseline; 1.8746x over previous
import functools

import jax
import jax.numpy as jnp
from jax import lax
from jax.experimental import pallas as pl
from jax.experimental.pallas import tpu as pltpu

N_DEV = 8


def kernel(x, w_mat):
    m, k_per = x.shape
    _, n = w_mat.shape
    m_per = m // N_DEV
    nh = n // 2

    def body(x_ref, w_ref, out_ref, buf_r, buf_l, send_r, recv_r,
             send_l, recv_l, cred_r, cred_l, out_sems):
        d = lax.axis_index("i")
        left = lax.rem(d + N_DEV - 1, N_DEV)
        right = lax.rem(d + 1, N_DEV)

        barrier = pltpu.get_barrier_semaphore()
        for nbr in (left, right):
            pl.semaphore_signal(
                barrier, inc=1, device_id=(nbr,),
                device_id_type=pl.DeviceIdType.MESH,
            )
        pl.semaphore_wait(barrier, 2)

        def gemm(c, lo):
            xc = x_ref[pl.ds(c * m_per, m_per), :]
            return jnp.dot(
                xc, w_ref[:, lo:lo + nh], preferred_element_type=jnp.float32
            )

        for h in range(N_DEV - 1):
            s = h % 2
            r = (h + 1) % 2
            c_r = lax.rem(d + (N_DEV - 1) - h, N_DEV)
            c_l = lax.rem(d + 1 + h, N_DEV)
            if h == 0:
                buf_r[s] = gemm(c_r, 0)
                buf_l[s] = gemm(c_l, nh)
            else:
                buf_r[s] = buf_r[s] + gemm(c_r, 0)
                buf_l[s] = buf_l[s] + gemm(c_l, nh)
            rdma_r = pltpu.make_async_remote_copy(
                src_ref=buf_r.at[s],
                dst_ref=buf_r.at[r],
                send_sem=send_r.at[s],
                recv_sem=recv_r.at[r],
                device_id=(right,),
                device_id_type=pl.DeviceIdType.MESH,
            )
            rdma_l = pltpu.make_async_remote_copy(
                src_ref=buf_l.at[s],
                dst_ref=buf_l.at[r],
                send_sem=send_l.at[s],
                recv_sem=recv_l.at[r],
                device_id=(left,),
                device_id_type=pl.DeviceIdType.MESH,
            )
            if h >= 1:
                pl.semaphore_wait(cred_r, 1)
                pl.semaphore_wait(cred_l, 1)
            rdma_r.start()
            rdma_l.start()
            rdma_r.wait()
            rdma_l.wait()
            if h < N_DEV - 2:
                pl.semaphore_signal(
                    cred_r, inc=1, device_id=(left,),
                    device_id_type=pl.DeviceIdType.MESH,
                )
                pl.semaphore_signal(
                    cred_l, inc=1, device_id=(right,),
                    device_id_type=pl.DeviceIdType.MESH,
                )

        buf_r[0] = buf_r[1] + gemm(d, 0)
        buf_l[0] = buf_l[1] + gemm(d, nh)
        copy_r = pltpu.make_async_copy(
            buf_r.at[0], out_ref.at[:, pl.ds(0, nh)], out_sems.at[0]
        )
        copy_l = pltpu.make_async_copy(
            buf_l.at[0], out_ref.at[:, pl.ds(nh, nh)], out_sems.at[1]
        )
        copy_r.start()
        copy_l.start()
        copy_r.wait()
        copy_l.wait()

        @functools.partial(
            pl.run_scoped, second_barrier=pltpu.SemaphoreType.REGULAR
        )
        def _(second_barrier):
            for nbr in (left, right):
                pl.semaphore_signal(
                    second_barrier, inc=1, device_id=(nbr,),
                    device_id_type=pl.DeviceIdType.MESH,
                )
            pl.semaphore_wait(second_barrier, 2)

    return pl.pallas_call(
        body,
        out_shape=jax.ShapeDtypeStruct((m_per, n), jnp.float32),
        in_specs=[
            pl.BlockSpec(memory_space=pltpu.VMEM),
            pl.BlockSpec(memory_space=pltpu.VMEM),
        ],
        out_specs=pl.BlockSpec(memory_space=pl.ANY),
        scratch_shapes=[
            pltpu.VMEM((2, m_per, nh), jnp.float32),
            pltpu.VMEM((2, m_per, nh), jnp.float32),
            pltpu.SemaphoreType.DMA((2,)),
            pltpu.SemaphoreType.DMA((2,)),
            pltpu.SemaphoreType.DMA((2,)),
            pltpu.SemaphoreType.DMA((2,)),
            pltpu.SemaphoreType.REGULAR,
            pltpu.SemaphoreType.REGULAR,
            pltpu.SemaphoreType.DMA((2,)),
        ],
        compiler_params=pltpu.CompilerParams(
            collective_id=0,
            vmem_limit_bytes=63 * 1024 * 1024,
        ),
    )(x, w_mat)


# device time: 665876 ns/iter; 2.0172x vs baseline; 1.0761x over previous
import functools

import jax
import jax.numpy as jnp
from jax import lax
from jax.experimental import pallas as pl
from jax.experimental.pallas import tpu as pltpu

N_DEV = 8
N_HOP = N_DEV - 1


def kernel(x, w_mat):
    m, k_per = x.shape
    _, n = w_mat.shape
    m_per = m // N_DEV
    nh = n // 2
    nq = n // 4

    def body(x_ref, w_ref, out_ref, buf_r, buf_l, send_r, recv_r,
             send_l, recv_l, cred_r, cred_l, out_sems):
        d = lax.axis_index("i")
        left = lax.rem(d + N_DEV - 1, N_DEV)
        right = lax.rem(d + 1, N_DEV)

        barrier = pltpu.get_barrier_semaphore()
        for nbr in (left, right):
            pl.semaphore_signal(
                barrier, inc=1, device_id=(nbr,),
                device_id_type=pl.DeviceIdType.MESH,
            )
        pl.semaphore_wait(barrier, 2)

        def gemm(c, lo):
            xc = x_ref[pl.ds(c * m_per, m_per), :]
            return jnp.dot(
                xc, w_ref[:, lo:lo + nq], preferred_element_type=jnp.float32
            )

        def mk_rdmas(h, q):
            s = h % 2
            r = (h + 1) % 2
            rdma_r = pltpu.make_async_remote_copy(
                src_ref=buf_r.at[s, :, pl.ds(q * nq, nq)],
                dst_ref=buf_r.at[r, :, pl.ds(q * nq, nq)],
                send_sem=send_r.at[s, q],
                recv_sem=recv_r.at[r, q],
                device_id=(right,),
                device_id_type=pl.DeviceIdType.MESH,
            )
            rdma_l = pltpu.make_async_remote_copy(
                src_ref=buf_l.at[s, :, pl.ds(q * nq, nq)],
                dst_ref=buf_l.at[r, :, pl.ds(q * nq, nq)],
                send_sem=send_l.at[s, q],
                recv_sem=recv_l.at[r, q],
                device_id=(left,),
                device_id_type=pl.DeviceIdType.MESH,
            )
            return rdma_r, rdma_l

        for h in range(N_HOP):
            s = h % 2
            c_r = lax.rem(d + N_DEV - 1 - h, N_DEV)
            c_l = lax.rem(d + 1 + h, N_DEV)
            for q in (0, 1):
                qs = pl.ds(q * nq, nq)
                if h == 0:
                    buf_r[s, :, qs] = gemm(c_r, q * nq)
                    buf_l[s, :, qs] = gemm(c_l, nh + q * nq)
                else:
                    prev_r, prev_l = mk_rdmas(h - 1, q)
                    prev_r.wait()
                    prev_l.wait()
                    pl.semaphore_signal(
                        cred_r.at[q], inc=1, device_id=(left,),
                        device_id_type=pl.DeviceIdType.MESH,
                    )
                    pl.semaphore_signal(
                        cred_l.at[q], inc=1, device_id=(right,),
                        device_id_type=pl.DeviceIdType.MESH,
                    )
                    buf_r[s, :, qs] = buf_r[s, :, qs] + gemm(c_r, q * nq)
                    buf_l[s, :, qs] = buf_l[s, :, qs] + gemm(c_l, nh + q * nq)
                rdma_r, rdma_l = mk_rdmas(h, q)
                if h >= 1:
                    pl.semaphore_wait(cred_r.at[q], 1)
                    pl.semaphore_wait(cred_l.at[q], 1)
                rdma_r.start()
                rdma_l.start()

        copies = []
        for q in (0, 1):
            qs = pl.ds(q * nq, nq)
            prev_r, prev_l = mk_rdmas(N_HOP - 1, q)
            prev_r.wait()
            prev_l.wait()
            buf_r[0, :, qs] = buf_r[1, :, qs] + gemm(d, q * nq)
            buf_l[0, :, qs] = buf_l[1, :, qs] + gemm(d, nh + q * nq)
            copy_r = pltpu.make_async_copy(
                buf_r.at[0, :, qs], out_ref.at[:, pl.ds(q * nq, nq)],
                out_sems.at[0, q],
            )
            copy_l = pltpu.make_async_copy(
                buf_l.at[0, :, qs], out_ref.at[:, pl.ds(nh + q * nq, nq)],
                out_sems.at[1, q],
            )
            copy_r.start()
            copy_l.start()
            copies += [copy_r, copy_l]
        for copy in copies:
            copy.wait()

        @functools.partial(
            pl.run_scoped, second_barrier=pltpu.SemaphoreType.REGULAR
        )
        def _(second_barrier):
            for nbr in (left, right):
                pl.semaphore_signal(
                    second_barrier, inc=1, device_id=(nbr,),
                    device_id_type=pl.DeviceIdType.MESH,
                )
            pl.semaphore_wait(second_barrier, 2)

    return pl.pallas_call(
        body,
        out_shape=jax.ShapeDtypeStruct((m_per, n), jnp.float32),
        in_specs=[
            pl.BlockSpec(memory_space=pltpu.VMEM),
            pl.BlockSpec(memory_space=pltpu.VMEM),
        ],
        out_specs=pl.BlockSpec(memory_space=pl.ANY),
        scratch_shapes=[
            pltpu.VMEM((2, m_per, nh), jnp.float32),
            pltpu.VMEM((2, m_per, nh), jnp.float32),
            pltpu.SemaphoreType.DMA((2, 2)),
            pltpu.SemaphoreType.DMA((2, 2)),
            pltpu.SemaphoreType.DMA((2, 2)),
            pltpu.SemaphoreType.DMA((2, 2)),
            pltpu.SemaphoreType.REGULAR((2,)),
            pltpu.SemaphoreType.REGULAR((2,)),
            pltpu.SemaphoreType.DMA((2, 2)),
        ],
        compiler_params=pltpu.CompilerParams(
            collective_id=0,
            vmem_limit_bytes=63 * 1024 * 1024,
        ),
    )(x, w_mat)


# device time: 665551 ns/iter; 2.0182x vs baseline; 1.0005x over previous
import functools

import jax
import jax.numpy as jnp
from jax import lax
from jax.experimental import pallas as pl
from jax.experimental.pallas import tpu as pltpu

N_DEV = 8
N_HOP = N_DEV - 1


def kernel(x, w_mat):
    m, k_per = x.shape
    _, n = w_mat.shape
    m_per = m // N_DEV
    nh = n // 2
    nq = n // 4

    def body(x_ref, w_ref, out_ref, buf_r, buf_l, send_r, recv_r,
             send_l, recv_l, cred_r, cred_l, out_sems):
        d = lax.axis_index("i")
        left = lax.rem(d + N_DEV - 1, N_DEV)
        right = lax.rem(d + 1, N_DEV)

        barrier = pltpu.get_barrier_semaphore()
        for nbr in (left, right):
            pl.semaphore_signal(
                barrier, inc=1, device_id=(nbr,),
                device_id_type=pl.DeviceIdType.MESH,
            )
        pl.semaphore_wait(barrier, 2)

        def gemm(c, lo, width=nq):
            xc = x_ref[pl.ds(c * m_per, m_per), :]
            return jnp.dot(
                xc, w_ref[:, lo:lo + width],
                preferred_element_type=jnp.float32,
            )

        def mk_rdmas(h, q):
            s = h % 2
            r = (h + 1) % 2
            rdma_r = pltpu.make_async_remote_copy(
                src_ref=buf_r.at[s, :, pl.ds(q * nq, nq)],
                dst_ref=buf_r.at[r, :, pl.ds(q * nq, nq)],
                send_sem=send_r.at[s, q],
                recv_sem=recv_r.at[r, q],
                device_id=(right,),
                device_id_type=pl.DeviceIdType.MESH,
            )
            rdma_l = pltpu.make_async_remote_copy(
                src_ref=buf_l.at[s, :, pl.ds(q * nq, nq)],
                dst_ref=buf_l.at[r, :, pl.ds(q * nq, nq)],
                send_sem=send_l.at[s, q],
                recv_sem=recv_l.at[r, q],
                device_id=(left,),
                device_id_type=pl.DeviceIdType.MESH,
            )
            return rdma_r, rdma_l

        for h in range(N_HOP):
            s = h % 2
            c_r = lax.rem(d + N_DEV - 1 - h, N_DEV)
            c_l = lax.rem(d + 1 + h, N_DEV)
            for q in (0, 1):
                qs = pl.ds(q * nq, nq)
                rdma_r, rdma_l = mk_rdmas(h, q)
                if h == 0:
                    buf_r[s, :, qs] = gemm(c_r, q * nq)
                    rdma_r.start()
                    buf_l[s, :, qs] = gemm(c_l, nh + q * nq)
                    rdma_l.start()
                else:
                    prev_r, prev_l = mk_rdmas(h - 1, q)
                    prev_r.wait()
                    prev_l.wait()
                    pl.semaphore_signal(
                        cred_r.at[q], inc=1, device_id=(left,),
                        device_id_type=pl.DeviceIdType.MESH,
                    )
                    pl.semaphore_signal(
                        cred_l.at[q], inc=1, device_id=(right,),
                        device_id_type=pl.DeviceIdType.MESH,
                    )
                    buf_r[s, :, qs] = buf_r[s, :, qs] + gemm(c_r, q * nq)
                    pl.semaphore_wait(cred_r.at[q], 1)
                    rdma_r.start()
                    buf_l[s, :, qs] = buf_l[s, :, qs] + gemm(c_l, nh + q * nq)
                    pl.semaphore_wait(cred_l.at[q], 1)
                    rdma_l.start()

        ne = n // 8
        copies = []
        for u in range(4):
            us = pl.ds(u * ne, ne)
            if u % 2 == 0:
                prev_r, prev_l = mk_rdmas(N_HOP - 1, u // 2)
                prev_r.wait()
                prev_l.wait()
            buf_r[0, :, us] = buf_r[1, :, us] + gemm(d, u * ne, ne)
            copy_r = pltpu.make_async_copy(
                buf_r.at[0, :, us], out_ref.at[:, pl.ds(u * ne, ne)],
                out_sems.at[0, u],
            )
            copy_r.start()
            buf_l[0, :, us] = buf_l[1, :, us] + gemm(d, nh + u * ne, ne)
            copy_l = pltpu.make_async_copy(
                buf_l.at[0, :, us], out_ref.at[:, pl.ds(nh + u * ne, ne)],
                out_sems.at[1, u],
            )
            copy_l.start()
            copies += [copy_r, copy_l]
        for copy in copies:
            copy.wait()

        @functools.partial(
            pl.run_scoped, second_barrier=pltpu.SemaphoreType.REGULAR
        )
        def _(second_barrier):
            for nbr in (left, right):
                pl.semaphore_signal(
                    second_barrier, inc=1, device_id=(nbr,),
                    device_id_type=pl.DeviceIdType.MESH,
                )
            pl.semaphore_wait(second_barrier, 2)

    return pl.pallas_call(
        body,
        out_shape=jax.ShapeDtypeStruct((m_per, n), jnp.float32),
        in_specs=[
            pl.BlockSpec(memory_space=pltpu.VMEM),
            pl.BlockSpec(memory_space=pltpu.VMEM),
        ],
        out_specs=pl.BlockSpec(memory_space=pl.ANY),
        scratch_shapes=[
            pltpu.VMEM((2, m_per, nh), jnp.float32),
            pltpu.VMEM((2, m_per, nh), jnp.float32),
            pltpu.SemaphoreType.DMA((2, 2)),
            pltpu.SemaphoreType.DMA((2, 2)),
            pltpu.SemaphoreType.DMA((2, 2)),
            pltpu.SemaphoreType.DMA((2, 2)),
            pltpu.SemaphoreType.REGULAR((2,)),
            pltpu.SemaphoreType.REGULAR((2,)),
            pltpu.SemaphoreType.DMA((2, 4)),
        ],
        compiler_params=pltpu.CompilerParams(
            collective_id=0,
            vmem_limit_bytes=63 * 1024 * 1024,
        ),
    )(x, w_mat)
